# Initial kernel scaffold; baseline (speedup 1.0000x reference)
import jax
import jax.numpy as jnp
from jax.experimental import pallas as pl


def kernel(x, adj_t, Wl0, bl0, Wr0, Wl1, bl1, Wr1, Wl2, bl2, Wr2):
    raise NotImplementedError("write your pallas kernel here")


# speedup vs baseline: 4.2405x; 4.2405x over previous
import functools

import jax
import jax.numpy as jnp
from jax import lax
from jax.experimental import pallas as pl
from jax.experimental.pallas import tpu as pltpu
from jax.experimental.pallas import tpu_sc as plsc

N_NODES = 10000
N_EDGES = 320000
NC = 2
NS = 16
EB = 80
RT = 624
TAIL = N_NODES - NS * RT

_mesh = plsc.VectorSubcoreMesh(core_axis_name="c", subcore_axis_name="s")


def _zero_stripe(z_hbm, sh, s):
    pltpu.sync_copy(z_hbm.at[pl.ds(0, RT)], sh.at[pl.ds(s * RT, RT)])

    @pl.when(s == NS - 1)
    def _():
        pltpu.sync_copy(z_hbm.at[pl.ds(0, TAIL)],
                        sh.at[pl.ds(NS * RT, TAIL)])


def _write_stripe(sh, out_hbm, c, s):
    pltpu.sync_copy(sh.at[pl.ds(s * RT, RT)],
                    out_hbm.at[c, pl.ds(s * RT, RT), :])

    @pl.when(s == NS - 1)
    def _():
        pltpu.sync_copy(sh.at[pl.ds(NS * RT, TAIL)],
                        out_hbm.at[c, pl.ds(NS * RT, TAIL), :])


def _edge_loop(tab_hbm, src_hbm, dst_hbm, agg_sh, sidx, didx, rows, sem,
               base, steps):
    def body(i, _):
        off = base + i * EB
        pltpu.sync_copy(src_hbm.at[pl.ds(off, EB)], sidx)
        pltpu.sync_copy(dst_hbm.at[pl.ds(off, EB)], didx)
        pltpu.async_copy(tab_hbm.at[sidx], rows, sem).wait()
        pltpu.sync_copy(rows, agg_sh.at[didx], add=True)
        return 0

    lax.fori_loop(0, steps, body, 0)


def _make_segsum_edge_split():
    epc = N_EDGES // NC
    ept = epc // NS
    steps = ept // EB

    scratch = [
        pltpu.VMEM((EB,), jnp.int32),
        pltpu.VMEM((EB,), jnp.int32),
        pltpu.VMEM((EB, 128), jnp.float32),
        pltpu.VMEM_SHARED((N_NODES, 128), jnp.float32),
        pltpu.SemaphoreType.DMA,
    ]

    def body(tab_hbm, src_hbm, dst_hbm, z128_hbm, out_hbm,
             sidx, didx, rows, agg_sh, sem):
        c = lax.axis_index("c")
        s = lax.axis_index("s")
        _zero_stripe(z128_hbm, agg_sh, s)
        plsc.subcore_barrier()
        base = c * epc + s * ept
        _edge_loop(tab_hbm, src_hbm, dst_hbm, agg_sh, sidx, didx, rows, sem,
                   base, steps)
        plsc.subcore_barrier()
        _write_stripe(agg_sh, out_hbm, c, s)

    return pl.kernel(
        body, out_type=jax.ShapeDtypeStruct((NC, N_NODES, 128), jnp.float32),
        mesh=_mesh, scratch_types=scratch)


def _make_deg():
    epc = N_EDGES // NC
    ept = epc // NS
    steps = ept // EB

    scratch = [
        pltpu.VMEM((EB,), jnp.int32),
        pltpu.VMEM((EB, 128), jnp.float32),
        pltpu.VMEM_SHARED((N_NODES, 128), jnp.float32),
    ]

    def body(dst_hbm, z128_hbm, ones_hbm, out_hbm, didx, ones_v, deg_sh):
        c = lax.axis_index("c")
        s = lax.axis_index("s")
        _zero_stripe(z128_hbm, deg_sh, s)
        pltpu.sync_copy(ones_hbm, ones_v)
        plsc.subcore_barrier()
        base = c * epc + s * ept

        def lbody(i, _):
            off = base + i * EB
            pltpu.sync_copy(dst_hbm.at[pl.ds(off, EB)], didx)
            pltpu.sync_copy(ones_v, deg_sh.at[didx], add=True)
            return 0

        lax.fori_loop(0, steps, lbody, 0)
        plsc.subcore_barrier()
        _write_stripe(deg_sh, out_hbm, c, s)

    return pl.kernel(
        body, out_type=jax.ShapeDtypeStruct((NC, N_NODES, 128), jnp.float32),
        mesh=_mesh, scratch_types=scratch)


def _make_segsum_channel_split():
    ept = N_EDGES // NS
    steps = ept // EB

    scratch = [
        pltpu.VMEM((EB,), jnp.int32),
        pltpu.VMEM((EB,), jnp.int32),
        pltpu.VMEM((EB, 128), jnp.float32),
        pltpu.VMEM_SHARED((N_NODES, 128), jnp.float32),
        pltpu.SemaphoreType.DMA,
    ]

    def body(t0_hbm, t1_hbm, src_hbm, dst_hbm, z128_hbm, out_hbm,
             sidx, didx, rows, agg_sh, sem):
        c = lax.axis_index("c")
        s = lax.axis_index("s")
        _zero_stripe(z128_hbm, agg_sh, s)
        plsc.subcore_barrier()
        base = s * ept

        @pl.when(c == 0)
        def _():
            _edge_loop(t0_hbm, src_hbm, dst_hbm, agg_sh, sidx, didx, rows,
                       sem, base, steps)

        @pl.when(c == 1)
        def _():
            _edge_loop(t1_hbm, src_hbm, dst_hbm, agg_sh, sidx, didx, rows,
                       sem, base, steps)

        plsc.subcore_barrier()
        _write_stripe(agg_sh, out_hbm, c, s)

    return pl.kernel(
        body, out_type=jax.ShapeDtypeStruct((NC, N_NODES, 128), jnp.float32),
        mesh=_mesh, scratch_types=scratch)


_segsum_edges = _make_segsum_edge_split()
_segsum_chans = _make_segsum_channel_split()
_deg_counts = _make_deg()


BN = 2000


def _recip_deg(degp_ref):
    deg = degp_ref[0, :, 0:1] + degp_ref[1, :, 0:1]
    return 1.0 / jnp.maximum(deg, 1.0)


def _dot(a, b):
    return jnp.dot(a, b, preferred_element_type=jnp.float32)


def _layer0_body(pa_ref, degp_ref, x_ref, wl_ref, bl_ref, wr_ref,
                 outa_ref, outb_ref):
    mean = (pa_ref[0] + pa_ref[1]) * _recip_deg(degp_ref)
    h = _dot(mean, wl_ref[...]) + bl_ref[...] + _dot(x_ref[...], wr_ref[...])
    h = jnp.maximum(h, 0.0)
    outa_ref[...] = h[:, :128]
    outb_ref[...] = h[:, 128:]


def _layer1_body(agg_ref, degp_ref, h1a_ref, h1b_ref, wla_ref, wlb_ref,
                 bl_ref, wra_ref, wrb_ref, wl2_ref,
                 outa_ref, outb_ref, outm_ref):
    recip = _recip_deg(degp_ref)
    h = (_dot(agg_ref[0] * recip, wla_ref[...])
         + _dot(agg_ref[1] * recip, wlb_ref[...])
         + bl_ref[...]
         + _dot(h1a_ref[...], wra_ref[...])
         + _dot(h1b_ref[...], wrb_ref[...]))
    h = jnp.maximum(h, 0.0)
    outa_ref[...] = h[:, :128]
    outb_ref[...] = h[:, 128:]
    outm_ref[...] = _dot(h, wl2_ref[...])


def _layer2_body(pm_ref, degp_ref, h2a_ref, h2b_ref, wra_ref, wrb_ref,
                 bl_ref, out_ref):
    mean_wl = (pm_ref[0] + pm_ref[1]) * _recip_deg(degp_ref)
    out_ref[...] = (mean_wl + bl_ref[...]
                    + _dot(h2a_ref[...], wra_ref[...])
                    + _dot(h2b_ref[...], wrb_ref[...]))


def _node_spec(ch):
    return pl.BlockSpec((NC, BN, ch), lambda i: (0, i, 0))


def _row_spec(ch):
    return pl.BlockSpec((BN, ch), lambda i: (i, 0))


def _full_spec(shape):
    n = len(shape)
    return pl.BlockSpec(shape, lambda i: (0,) * n)


_GRID = (N_NODES // BN,)


def _layer0(pa, degp, x, wl, bl, wr):
    return pl.pallas_call(
        _layer0_body,
        grid=_GRID,
        in_specs=[_node_spec(128), _node_spec(128), _row_spec(128),
                  _full_spec(wl.shape), _full_spec(bl.shape),
                  _full_spec(wr.shape)],
        out_specs=[_row_spec(128), _row_spec(128)],
        out_shape=[jax.ShapeDtypeStruct((N_NODES, 128), jnp.float32)] * 2,
    )(pa, degp, x, wl, bl, wr)


def _layer1(agg, degp, h1a, h1b, wla, wlb, bl, wra, wrb, wl2):
    return pl.pallas_call(
        _layer1_body,
        grid=_GRID,
        in_specs=[_node_spec(128), _node_spec(128), _row_spec(128),
                  _row_spec(128), _full_spec(wla.shape),
                  _full_spec(wlb.shape), _full_spec(bl.shape),
                  _full_spec(wra.shape), _full_spec(wrb.shape),
                  _full_spec(wl2.shape)],
        out_specs=[_row_spec(128), _row_spec(128), _row_spec(128)],
        out_shape=[jax.ShapeDtypeStruct((N_NODES, 128), jnp.float32)] * 3,
    )(agg, degp, h1a, h1b, wla, wlb, bl, wra, wrb, wl2)


def _layer2(pm, degp, h2a, h2b, wra, wrb, bl):
    return pl.pallas_call(
        _layer2_body,
        grid=_GRID,
        in_specs=[_node_spec(128), _node_spec(128), _row_spec(128),
                  _row_spec(128), _full_spec(wra.shape),
                  _full_spec(wrb.shape), _full_spec(bl.shape)],
        out_specs=pl.BlockSpec((BN, 128), lambda i: (i, 0)),
        out_shape=jax.ShapeDtypeStruct((N_NODES, 128), jnp.float32),
    )(pm, degp, h2a, h2b, wra, wrb, bl)


@jax.jit
def kernel(x, adj_t, Wl0, bl0, Wr0, Wl1, bl1, Wr1, Wl2, bl2, Wr2):
    src = adj_t[0].astype(jnp.int32)
    dst = adj_t[1].astype(jnp.int32)
    z128 = jnp.zeros((RT, 128), jnp.float32)
    ones128 = jnp.ones((EB, 128), jnp.float32)

    degp = _deg_counts(dst, z128, ones128)
    pa = _segsum_edges(x, src, dst, z128)
    h1a, h1b = _layer0(pa, degp, x, Wl0, bl0.reshape(1, -1), Wr0)

    agg1 = _segsum_chans(h1a, h1b, src, dst, z128)
    h2a, h2b, m = _layer1(agg1, degp, h1a, h1b, Wl1[:128], Wl1[128:],
                          bl1.reshape(1, -1), Wr1[:128], Wr1[128:], Wl2)

    pm = _segsum_edges(m, src, dst, z128)
    out = _layer2(pm, degp, h2a, h2b, Wr2[:128], Wr2[128:],
                  bl2.reshape(1, -1))
    return out
